# Initial kernel scaffold; baseline (speedup 1.0000x reference)
import jax
import jax.numpy as jnp
from jax.experimental import pallas as pl


def kernel(x, edge_index, Wl0, bl0, Wr0, Wl1, bl1, Wr1):
    raise NotImplementedError("write your pallas kernel here")


# speedup vs baseline: 4.4111x; 4.4111x over previous
import functools

import jax
import jax.numpy as jnp
from jax import lax
from jax.experimental import pallas as pl
from jax.experimental.pallas import tpu as pltpu
from jax.experimental.pallas import tpu_sc as plsc

NC = 2
NS = 16
NW = NC * NS
L = 16

K = 128
ACC_ROWS = 10112
CNT_N = 10240


@functools.lru_cache(maxsize=None)
def _make_agg_kernel(n, d, e_pad, with_cnt):
  epw = e_pad // NW
  n_chunks = epw // K
  zpw = ACC_ROWS // NS
  cpw = CNT_N // NS

  mesh = plsc.VectorSubcoreMesh(core_axis_name="c", subcore_axis_name="s",
                                num_cores=NC, num_subcores=NS)
  out_type = [jax.ShapeDtypeStruct((NC, ACC_ROWS, d), jnp.float32)]
  scratch = [
      pltpu.VMEM((K,), jnp.int32),
      pltpu.VMEM((K,), jnp.int32),
      pltpu.VMEM((K, d), jnp.float32),
      pltpu.VMEM((8, d), jnp.float32),
      pltpu.VMEM_SHARED((ACC_ROWS, d), jnp.float32),
      pltpu.SemaphoreType.DMA,
  ]
  if with_cnt:
    out_type.append(jax.ShapeDtypeStruct((NC, CNT_N), jnp.float32))
    scratch += [
        pltpu.VMEM((CNT_N,), jnp.float32),
        pltpu.VMEM((NS, cpw), jnp.float32),
        pltpu.VMEM_SHARED((NS, CNT_N), jnp.float32),
    ]

  def body(h_hbm, src_hbm, dst_hbm, out_agg, *rest):
    if with_cnt:
      (out_cnt, src_v, dst_v, rows_v, zbuf, acc, sem,
       cnt_l, red_buf, cnt_stage) = rest
    else:
      (src_v, dst_v, rows_v, zbuf, acc, sem) = rest
    c = lax.axis_index("c")
    s = lax.axis_index("s")
    wid = s * NC + c

    zv = jnp.zeros((L,), jnp.float32)

    for i in range(8):
      for j in range(d // L):
        zbuf[i, pl.ds(j * L, L)] = zv

    def zero_acc(i, carry):
      pltpu.sync_copy(zbuf, acc.at[pl.ds(s * zpw + i * 8, 8)])
      return carry
    lax.fori_loop(0, zpw // 8, zero_acc, 0)

    if with_cnt:
      def zero_cnt(i, carry):
        cnt_l[pl.ds(i * L, L)] = zv
        return carry
      lax.fori_loop(0, CNT_N // L, zero_cnt, 0)
    plsc.subcore_barrier()

    base = wid * epw
    ones = jnp.full((L,), 1.0, jnp.float32)

    def step(i, carry):
      off = pl.multiple_of(base + i * K, K)
      pltpu.sync_copy(src_hbm.at[pl.ds(off, K)], src_v)
      pltpu.sync_copy(dst_hbm.at[pl.ds(off, K)], dst_v)
      pltpu.async_copy(h_hbm.at[src_v], rows_v, sem).wait()
      if with_cnt:
        for j in range(K // L):
          dv = dst_v[pl.ds(j * L, L)]
          plsc.addupdate_scatter(cnt_l, [dv], ones)
      pltpu.sync_copy(rows_v, acc.at[dst_v], add=True)
      return carry
    lax.fori_loop(0, n_chunks, step, 0)

    if with_cnt:
      pltpu.sync_copy(cnt_l, cnt_stage.at[s])
    plsc.subcore_barrier()

    o_off = pl.multiple_of(s * zpw, 8)
    pltpu.sync_copy(acc.at[pl.ds(o_off, zpw)],
                    out_agg.at[c, pl.ds(o_off, zpw)])
    if with_cnt:
      c_off = pl.multiple_of(s * cpw, 8)
      pltpu.sync_copy(cnt_stage.at[:, pl.ds(c_off, cpw)], red_buf)
      def red_step(i, carry):
        tot = red_buf[0, pl.ds(i * L, L)]
        for r in range(1, NS):
          tot = tot + red_buf[r, pl.ds(i * L, L)]
        cnt_l[pl.ds(i * L, L)] = tot
        return carry
      lax.fori_loop(0, cpw // L, red_step, 0)
      pltpu.sync_copy(cnt_l.at[pl.ds(0, cpw)],
                      out_cnt.at[c, pl.ds(c_off, cpw)])

  return pl.kernel(
      body, out_type=out_type, mesh=mesh, scratch_types=scratch,
      compiler_params=pltpu.CompilerParams(needs_layout_passes=False))


def _layer_tc(p0, p1, c0, c1, h, Wl, bl, Wr, relu, resid):
  n, d = h.shape
  R = 400
  grid = (n // R,)

  def body(p0_ref, p1_ref, c0_ref, c1_ref, h_ref, wl_ref, bl_ref, wr_ref,
           *rest):
    if resid is not None:
      x_ref, o_ref = rest
    else:
      (o_ref,) = rest
    p = p0_ref[...] + p1_ref[...]
    cb = c0_ref[0] + c1_ref[0]
    ones_row = jnp.ones((1, d), jnp.float32)
    cc = lax.dot_general(cb, ones_row, (((0,), (0,)), ((), ())),
                         preferred_element_type=jnp.float32)
    aggm = p / jnp.maximum(cc, 1.0)
    y = lax.dot_general(aggm, wl_ref[...], (((1,), (1,)), ((), ())),
                        preferred_element_type=jnp.float32)
    y = y + lax.dot_general(h_ref[...], wr_ref[...], (((1,), (1,)), ((), ())),
                            preferred_element_type=jnp.float32)
    y = y + bl_ref[...]
    if relu:
      y = jnp.maximum(y, 0.0)
    if resid is not None:
      y = y + x_ref[...]
    o_ref[...] = y

  in_specs = [
      pl.BlockSpec((R, d), lambda j: (j, 0)),
      pl.BlockSpec((R, d), lambda j: (j, 0)),
      pl.BlockSpec((1, 1, R), lambda j: (j, 0, 0)),
      pl.BlockSpec((1, 1, R), lambda j: (j, 0, 0)),
      pl.BlockSpec((R, d), lambda j: (j, 0)),
      pl.BlockSpec((d, d), lambda j: (0, 0)),
      pl.BlockSpec((1, d), lambda j: (0, 0)),
      pl.BlockSpec((d, d), lambda j: (0, 0)),
  ]
  args = [p0, p1, c0, c1, h, Wl, bl.reshape(1, d), Wr]
  if resid is not None:
    in_specs.append(pl.BlockSpec((R, d), lambda j: (j, 0)))
    args.append(resid)
  return pl.pallas_call(
      body, grid=grid, in_specs=in_specs,
      out_specs=pl.BlockSpec((R, d), lambda j: (j, 0)),
      out_shape=jax.ShapeDtypeStruct((n, d), jnp.float32),
  )(*args)


def kernel(x, edge_index, Wl0, bl0, Wr0, Wl1, bl1, Wr1):
  n, d = x.shape
  e = edge_index.shape[1]
  e_pad = -(-e // (NW * K)) * (NW * K)
  pad = e_pad - e
  src = jnp.concatenate([edge_index[0], jnp.zeros((pad,), jnp.int32)])
  dst = jnp.concatenate([edge_index[1], jnp.full((pad,), n, jnp.int32)])

  agg0_k = _make_agg_kernel(n, d, e_pad, True)
  agg1_k = _make_agg_kernel(n, d, e_pad, False)

  part0, cnt = agg0_k(x, src, dst)
  part0 = part0[:, :n]
  cnt_flat = cnt[:, :n]
  c0 = cnt_flat[0].reshape(-1, 1, 400)
  c1 = cnt_flat[1].reshape(-1, 1, 400)

  h1 = _layer_tc(part0[0], part0[1], c0, c1, x, Wl0, bl0, Wr0,
                 relu=True, resid=None)
  part1 = agg1_k(h1, src, dst)
  if isinstance(part1, (list, tuple)):
    part1 = part1[0]
  part1 = part1[:, :n]
  out = _layer_tc(part1[0], part1[1], c0, c1, h1, Wl1, bl1, Wr1,
                  relu=False, resid=x)
  return out
